# baseline (speedup 1.0000x reference)
import functools
import math

import jax
import jax.numpy as jnp
from jax import lax
from jax.experimental import pallas as pl
from jax.experimental.pallas import tpu as pltpu
from jax.experimental.pallas import tpu_sc as plsc

_HEADS = 16


def _qkv_body(xq_ref, xk_ref, xv_ref, wq_ref, wk_ref, wv_ref, b_ref,
              oq_ref, ok_ref, ov_ref, *, q_scale):
    def proj(x_ref, w_ref, o_ref, brow, scale):
        acc = jnp.dot(
            x_ref[...].astype(jnp.bfloat16), w_ref[...],
            preferred_element_type=jnp.float32,
        )
        o_ref[...] = ((acc + b_ref[brow][None, :]) * scale).astype(jnp.bfloat16)

    proj(xq_ref, wq_ref, oq_ref, 0, q_scale)
    proj(xk_ref, wk_ref, ok_ref, 1, 1.0)
    proj(xv_ref, wv_ref, ov_ref, 2, 1.0)


def _qkv(xq, xk, xv, wq, wk, wv, bq, bk, bv, q_scale, bm=512):
    m, k = xq.shape
    n = wq.shape[1]
    bm = min(bm, m)
    bias3 = jnp.stack([bq, bk, bv])
    xspec = pl.BlockSpec((bm, k), lambda i: (i, 0))
    wspec = pl.BlockSpec((k, n), lambda i: (0, 0))
    ospec = pl.BlockSpec((bm, n), lambda i: (i, 0))
    shp = jax.ShapeDtypeStruct((m, n), jnp.bfloat16)
    return pl.pallas_call(
        functools.partial(_qkv_body, q_scale=q_scale),
        grid=(m // bm,),
        in_specs=[xspec, xspec, xspec, wspec, wspec, wspec,
                  pl.BlockSpec((3, n), lambda i: (0, 0))],
        out_specs=[ospec, ospec, ospec],
        out_shape=[shp, shp, shp],
        compiler_params=pltpu.CompilerParams(
            dimension_semantics=("arbitrary",)
        ),
    )(xq, xk, xv, wq, wk, wv, bias3)


def _flash_body(q_ref, k_ref, v_ref, ct_ref, w_ref, b_ref, o_ref,
                acc_ref, den_ref, *, nk, d):
    kb = pl.program_id(2)

    @pl.when(kb == 0)
    def _init():
        acc_ref[...] = jnp.zeros_like(acc_ref)
        den_ref[...] = jnp.zeros_like(den_ref)

    ct = ct_ref[...]
    for h in range(_HEADS):
        qh = q_ref[0, :, h * d:(h + 1) * d]
        kh = k_ref[0, :, h * d:(h + 1) * d]
        vh = v_ref[0, :, h * d:(h + 1) * d]
        st = lax.dot_general(
            kh, qh, (((1,), (1,)), ((), ())), preferred_element_type=jnp.float32
        )
        pf = ct * jax.lax.exp2(st)
        den_ref[h, :] += jnp.sum(pf, axis=0)
        acc_ref[h, :, :] += lax.dot_general(
            vh, pf.astype(jnp.bfloat16), (((0,), (0,)), ((), ())),
            preferred_element_type=jnp.float32,
        )

    @pl.when(kb == nk - 1)
    def _emit():
        blocks = []
        for h in range(_HEADS):
            den = den_ref[h, :]
            rden = jnp.where(den > 0, 1.0 / den, 0.0)[None, :]
            blocks.append((acc_ref[h] * rden).astype(jnp.bfloat16))
        att = jnp.concatenate(blocks, axis=0)
        o_ref[0] = (
            lax.dot_general(
                att, w_ref[...], (((0,), (0,)), ((), ())),
                preferred_element_type=jnp.float32,
            )
            + b_ref[0]
        )


def _flash(lq, lk, lv, counts_t, w_fc, b_fc, bq=2048, bk=512):
    b, l, mdl = lq.shape
    d = mdl // _HEADS
    bq = min(bq, l)
    bk = min(bk, l)
    nk = l // bk
    n_out = w_fc.shape[1]
    return pl.pallas_call(
        functools.partial(_flash_body, nk=nk, d=d),
        grid=(b, l // bq, nk),
        in_specs=[
            pl.BlockSpec((1, bq, mdl), lambda n, i, j: (n, i, 0)),
            pl.BlockSpec((1, bk, mdl), lambda n, i, j: (n, j, 0)),
            pl.BlockSpec((1, bk, mdl), lambda n, i, j: (n, j, 0)),
            pl.BlockSpec((bk, bq), lambda n, i, j: (j, i)),
            pl.BlockSpec((mdl, n_out), lambda n, i, j: (0, 0)),
            pl.BlockSpec((1, n_out), lambda n, i, j: (0, 0)),
        ],
        out_specs=pl.BlockSpec((1, bq, n_out), lambda n, i, j: (n, i, 0)),
        out_shape=jax.ShapeDtypeStruct((b, l, n_out), jnp.float32),
        scratch_shapes=[
            pltpu.VMEM((_HEADS, d, bq), jnp.float32),
            pltpu.VMEM((_HEADS, bq), jnp.float32),
        ],
        compiler_params=pltpu.CompilerParams(
            dimension_semantics=("parallel", "parallel", "arbitrary")
        ),
    )(lq, lk, lv, counts_t, w_fc, b_fc.reshape(1, n_out))


def _build_counts(a0, a1, l):
    e = a0.shape[0]
    nsub = 16
    chunk = e // nsub
    blk_rows = l // 4
    blk_elems = blk_rows * l
    slice_elems = blk_elems // nsub
    ngroups = chunk // 128
    mesh = plsc.VectorSubcoreMesh(core_axis_name="c", subcore_axis_name="s")

    @functools.partial(
        pl.kernel,
        mesh=mesh,
        out_type=jax.ShapeDtypeStruct((l * l,), jnp.float32),
        scratch_types=[
            pltpu.VMEM_SHARED((blk_elems + 256,), jnp.float32),
            pltpu.VMEM((chunk,), jnp.int32),
            pltpu.VMEM((chunk,), jnp.int32),
            pltpu.VMEM((ngroups, 128), jnp.int32),
            pltpu.VMEM((128,), jnp.float32),
            pltpu.VMEM((8192,), jnp.float32),
            pltpu.SemaphoreType.DMA,
        ],
    )
    def build(a0_hbm, a1_hbm, out_hbm, smem, a0v, a1v, idxb, ones_v, zbuf, sem):
        cid = lax.axis_index("c")
        sid = lax.axis_index("s")
        ecpy0 = pltpu.async_copy(a0_hbm.at[pl.ds(sid * chunk, chunk)], a0v, sem)
        ecpy1 = pltpu.async_copy(a1_hbm.at[pl.ds(sid * chunk, chunk)], a1v, sem)
        for i in range(8):
            ones_v[pl.ds(i * 16, 16)] = jnp.ones((16,), jnp.float32)
        for i in range(512):
            zbuf[pl.ds(i * 16, 16)] = jnp.zeros((16,), jnp.float32)
        ecpy0.wait()
        ecpy1.wait()
        lanes = lax.broadcasted_iota(jnp.int32, (16,), 0)
        trash = blk_elems + sid * 16 + lanes
        slice_off = sid * slice_elems
        nzero = slice_elems // 8192
        for bi in range(2):
            blk = cid * 2 + bi
            base = blk * blk_rows
            zcpys = [
                pltpu.async_copy(
                    zbuf, smem.at[pl.ds(slice_off + j * 8192, 8192)], sem
                )
                for j in range(nzero)
            ]
            for cpy in zcpys:
                cpy.wait()
            plsc.subcore_barrier()

            scpys = []
            for g in range(ngroups):
                for t in range(8):
                    off = g * 128 + t * 16
                    av0 = a0v[pl.ds(off, 16)]
                    av1 = a1v[pl.ds(off, 16)]
                    inb = (av0 >= base) & (av0 < base + blk_rows)
                    idx = jnp.where(inb, (av0 - base) * l + av1, trash)
                    idxb[g, pl.ds(t * 16, 16)] = idx
                scpys.append(
                    pltpu.async_copy(
                        ones_v, smem.at[idxb.at[g]], sem, add=True
                    )
                )
            for cpy in scpys:
                cpy.wait()
            plsc.subcore_barrier()
            pltpu.sync_copy(
                smem.at[pl.ds(slice_off, slice_elems)],
                out_hbm.at[pl.ds(blk * blk_elems + slice_off, slice_elems)],
            )
            plsc.subcore_barrier()

    return build(a0, a1)


def kernel(queries, keys, values, adj, Wq, bq, Wk, bk, Wv, bv, Wfc, bfc):
    b, l, cin = queries.shape
    mdl = Wq.shape[1]
    d = mdl // _HEADS
    temp = math.log2(math.e) / math.sqrt(d)

    counts_t = _build_counts(adj[1], adj[0], l).reshape(l, l)

    lq, lk, lv = _qkv(queries.reshape(b * l, cin), keys.reshape(b * l, cin),
                      values.reshape(b * l, cin),
                      Wq.astype(jnp.bfloat16), Wk.astype(jnp.bfloat16),
                      Wv.astype(jnp.bfloat16), bq, bk, bv, temp)

    return _flash(lq.reshape(b, l, mdl), lk.reshape(b, l, mdl),
                  lv.reshape(b, l, mdl), counts_t,
                  Wfc.astype(jnp.bfloat16), bfc)


# speedup vs baseline: 1.0197x; 1.0197x over previous
import functools
import math

import jax
import jax.numpy as jnp
from jax import lax
from jax.experimental import pallas as pl
from jax.experimental.pallas import tpu as pltpu
from jax.experimental.pallas import tpu_sc as plsc

_HEADS = 16


def _qkv_body(xq_ref, xk_ref, xv_ref, wq_ref, wk_ref, wv_ref, b_ref,
              oq_ref, ok_ref, ov_ref, *, q_scale):
    def proj(x_ref, w_ref, o_ref, brow, scale):
        acc = jnp.dot(
            x_ref[...].astype(jnp.bfloat16), w_ref[...].astype(jnp.bfloat16),
            preferred_element_type=jnp.float32,
        )
        o_ref[...] = ((acc + b_ref[brow][None, :]) * scale).astype(jnp.bfloat16)

    proj(xq_ref, wq_ref, oq_ref, 0, q_scale)
    proj(xk_ref, wk_ref, ok_ref, 1, 1.0)
    proj(xv_ref, wv_ref, ov_ref, 2, 1.0)


def _qkv(xq, xk, xv, wq, wk, wv, bq, bk, bv, q_scale, bm=512):
    m, k = xq.shape
    n = wq.shape[1]
    bm = min(bm, m)
    bias3 = jnp.stack([bq, bk, bv])
    xspec = pl.BlockSpec((bm, k), lambda i: (i, 0))
    wspec = pl.BlockSpec((k, n), lambda i: (0, 0))
    ospec = pl.BlockSpec((bm, n), lambda i: (i, 0))
    shp = jax.ShapeDtypeStruct((m, n), jnp.bfloat16)
    return pl.pallas_call(
        functools.partial(_qkv_body, q_scale=q_scale),
        grid=(m // bm,),
        in_specs=[xspec, xspec, xspec, wspec, wspec, wspec,
                  pl.BlockSpec((3, n), lambda i: (0, 0))],
        out_specs=[ospec, ospec, ospec],
        out_shape=[shp, shp, shp],
        compiler_params=pltpu.CompilerParams(
            dimension_semantics=("arbitrary",)
        ),
    )(xq, xk, xv, wq, wk, wv, bias3)


def _flash_body(q_ref, k_ref, v_ref, ct_ref, w_ref, b_ref, o_ref,
                acc_ref, den_ref, *, nk, d):
    kb = pl.program_id(2)

    @pl.when(kb == 0)
    def _init():
        acc_ref[...] = jnp.zeros_like(acc_ref)
        den_ref[...] = jnp.zeros_like(den_ref)

    ct = ct_ref[...]
    for h in range(_HEADS):
        qh = q_ref[0, :, h * d:(h + 1) * d]
        kh = k_ref[0, :, h * d:(h + 1) * d]
        vh = v_ref[0, :, h * d:(h + 1) * d]
        st = lax.dot_general(
            kh, qh, (((1,), (1,)), ((), ())), preferred_element_type=jnp.float32
        )
        pf = ct * jax.lax.exp2(st)
        den_ref[h, :] += jnp.sum(pf, axis=0)
        acc_ref[h, :, :] += lax.dot_general(
            vh, pf.astype(jnp.bfloat16), (((0,), (0,)), ((), ())),
            preferred_element_type=jnp.float32,
        )

    @pl.when(kb == nk - 1)
    def _emit():
        blocks = []
        for h in range(_HEADS):
            den = den_ref[h, :]
            rden = jnp.where(den > 0, 1.0 / den, 0.0)[None, :]
            blocks.append((acc_ref[h] * rden).astype(jnp.bfloat16))
        att = jnp.concatenate(blocks, axis=0)
        o_ref[0] = (
            lax.dot_general(
                att, w_ref[...], (((0,), (0,)), ((), ())),
                preferred_element_type=jnp.float32,
            )
            + b_ref[0]
        )


def _flash(lq, lk, lv, counts_t, w_fc, b_fc, bq=2048, bk=512):
    b, l, mdl = lq.shape
    d = mdl // _HEADS
    bq = min(bq, l)
    bk = min(bk, l)
    nk = l // bk
    n_out = w_fc.shape[1]
    return pl.pallas_call(
        functools.partial(_flash_body, nk=nk, d=d),
        grid=(b, l // bq, nk),
        in_specs=[
            pl.BlockSpec((1, bq, mdl), lambda n, i, j: (n, i, 0)),
            pl.BlockSpec((1, bk, mdl), lambda n, i, j: (n, j, 0)),
            pl.BlockSpec((1, bk, mdl), lambda n, i, j: (n, j, 0)),
            pl.BlockSpec((bk, bq), lambda n, i, j: (j, i)),
            pl.BlockSpec((mdl, n_out), lambda n, i, j: (0, 0)),
            pl.BlockSpec((1, n_out), lambda n, i, j: (0, 0)),
        ],
        out_specs=pl.BlockSpec((1, bq, n_out), lambda n, i, j: (n, i, 0)),
        out_shape=jax.ShapeDtypeStruct((b, l, n_out), jnp.float32),
        scratch_shapes=[
            pltpu.VMEM((_HEADS, d, bq), jnp.float32),
            pltpu.VMEM((_HEADS, bq), jnp.float32),
        ],
        compiler_params=pltpu.CompilerParams(
            dimension_semantics=("parallel", "parallel", "arbitrary")
        ),
    )(lq, lk, lv, counts_t, w_fc, b_fc.reshape(1, n_out))


def _build_counts(a0, a1, l):
    e = a0.shape[0]
    nsub = 16
    chunk = e // nsub
    blk_rows = l // 4
    blk_elems = blk_rows * l
    slice_elems = blk_elems // nsub
    ngroups = chunk // 128
    mesh = plsc.VectorSubcoreMesh(core_axis_name="c", subcore_axis_name="s")

    @functools.partial(
        pl.kernel,
        mesh=mesh,
        out_type=jax.ShapeDtypeStruct((l * l,), jnp.float32),
        scratch_types=[
            pltpu.VMEM_SHARED((blk_elems + 256,), jnp.float32),
            pltpu.VMEM((chunk,), jnp.int32),
            pltpu.VMEM((chunk,), jnp.int32),
            pltpu.VMEM((ngroups, 128), jnp.int32),
            pltpu.VMEM((128,), jnp.float32),
            pltpu.VMEM((8192,), jnp.float32),
            pltpu.SemaphoreType.DMA,
        ],
    )
    def build(a0_hbm, a1_hbm, out_hbm, smem, a0v, a1v, idxb, ones_v, zbuf, sem):
        cid = lax.axis_index("c")
        sid = lax.axis_index("s")
        ecpy0 = pltpu.async_copy(a0_hbm.at[pl.ds(sid * chunk, chunk)], a0v, sem)
        ecpy1 = pltpu.async_copy(a1_hbm.at[pl.ds(sid * chunk, chunk)], a1v, sem)
        for i in range(8):
            ones_v[pl.ds(i * 16, 16)] = jnp.ones((16,), jnp.float32)
        for i in range(512):
            zbuf[pl.ds(i * 16, 16)] = jnp.zeros((16,), jnp.float32)
        ecpy0.wait()
        ecpy1.wait()
        lanes = lax.broadcasted_iota(jnp.int32, (16,), 0)
        trash = blk_elems + sid * 16 + lanes
        slice_off = sid * slice_elems
        nzero = slice_elems // 8192
        for bi in range(2):
            blk = cid * 2 + bi
            base = blk * blk_rows
            zcpys = [
                pltpu.async_copy(
                    zbuf, smem.at[pl.ds(slice_off + j * 8192, 8192)], sem
                )
                for j in range(nzero)
            ]
            for cpy in zcpys:
                cpy.wait()
            plsc.subcore_barrier()

            scpys = []
            for g in range(ngroups):
                for t in range(8):
                    off = g * 128 + t * 16
                    av0 = a0v[pl.ds(off, 16)]
                    av1 = a1v[pl.ds(off, 16)]
                    inb = (av0 >= base) & (av0 < base + blk_rows)
                    idx = jnp.where(inb, (av0 - base) * l + av1, trash)
                    idxb[g, pl.ds(t * 16, 16)] = idx
                scpys.append(
                    pltpu.async_copy(
                        ones_v, smem.at[idxb.at[g]], sem, add=True
                    )
                )
            for cpy in scpys:
                cpy.wait()
            plsc.subcore_barrier()
            pltpu.sync_copy(
                smem.at[pl.ds(slice_off, slice_elems)],
                out_hbm.at[pl.ds(blk * blk_elems + slice_off, slice_elems)],
            )
            plsc.subcore_barrier()

    return build(a0, a1)


def kernel(queries, keys, values, adj, Wq, bq, Wk, bk, Wv, bv, Wfc, bfc):
    b, l, cin = queries.shape
    mdl = Wq.shape[1]
    d = mdl // _HEADS
    temp = math.log2(math.e) / math.sqrt(d)

    counts_t = _build_counts(adj[1], adj[0], l).reshape(l, l)

    lq, lk, lv = _qkv(queries.reshape(b * l, cin), keys.reshape(b * l, cin),
                      values.reshape(b * l, cin), Wq, Wk, Wv,
                      bq, bk, bv, temp)

    return _flash(lq.reshape(b, l, mdl), lk.reshape(b, l, mdl),
                  lv.reshape(b, l, mdl), counts_t,
                  Wfc.astype(jnp.bfloat16), bfc)
